# Initial kernel scaffold; baseline (speedup 1.0000x reference)
import jax
import jax.numpy as jnp
from jax.experimental import pallas as pl


def kernel(x_l, x_r, v_feature, adj, g_feature, emb_table, conv_w, conv_b, w2v_w0, w2v_b0, w2v_w1, w2v_b1, vf_w0, vf_b0, vf_w1, vf_b1, reg_w0, reg_b0, reg_w1, reg_b1):
    raise NotImplementedError("write your pallas kernel here")


# speedup vs baseline: 5.1889x; 5.1889x over previous
import functools

import jax
import jax.numpy as jnp
from jax import lax
from jax.experimental import pallas as pl
from jax.experimental.pallas import tpu as pltpu
from jax.experimental.pallas import tpu_sc as plsc

V = 10000
SEQ = 50
VOCAB = 30000
NFILT = 32

_NC = 2
_NS = 16
_NW = _NC * _NS
_NN = 2 * V
_NNP = 20480
_PW = _NNP // _NW
_CH = 32
_NCHUNK = _PW // _CH
_PAIRS = _CH // 2


def _proj_body(emb_ref, cwt_ref, cb_ref, out_ref):
    out_ref[:, :] = (
        jnp.dot(emb_ref[:, :], cwt_ref[:, :], preferred_element_type=jnp.float32)
        + cb_ref[:, :]
    )


def _project_table(emb_table, conv_w, conv_b):
    return pl.pallas_call(
        _proj_body,
        out_shape=jax.ShapeDtypeStruct((VOCAB, NFILT), jnp.float32),
    )(emb_table, conv_w.T, conv_b.reshape(1, NFILT))


def _enc_body(table_hbm, idx_hbm, out_hbm, idx_v, rows_v, out_v, sem):
    wid = lax.axis_index("s") * _NC + lax.axis_index("c")
    pair_base = wid * (_PW // 2)
    node_base = wid * _PW

    def chunk_body(g, carry):
        pltpu.sync_copy(idx_hbm.at[pl.ds(pair_base + g * _PAIRS, _PAIRS)], idx_v)
        copies = []
        for j in range(_PAIRS):
            copies.append(
                pltpu.async_copy(
                    table_hbm.at[idx_v.at[j]],
                    rows_v.at[pl.ds(j * 2 * SEQ, 2 * SEQ)],
                    sem,
                )
            )
        for c in copies:
            c.wait()

        def node_body(n, c2):
            acc0 = jnp.zeros((16,), jnp.float32)
            acc1 = jnp.zeros((16,), jnp.float32)
            for s in range(SEQ):
                r = n * SEQ + s
                acc0 = jnp.maximum(acc0, rows_v[r, pl.ds(0, 16)])
                acc1 = jnp.maximum(acc1, rows_v[r, pl.ds(16, 16)])
            out_v[n, pl.ds(0, 16)] = acc0
            out_v[n, pl.ds(16, 16)] = acc1
            return c2

        lax.fori_loop(0, _CH, node_body, 0)
        pltpu.sync_copy(out_v, out_hbm.at[pl.ds(node_base + g * _CH, _CH)])
        return carry

    lax.fori_loop(0, _NCHUNK, chunk_body, 0)


def _encode_sc(table, idx_pairs):
    mesh = plsc.VectorSubcoreMesh(core_axis_name="c", subcore_axis_name="s")
    k = functools.partial(
        pl.kernel,
        mesh=mesh,
        compiler_params=pltpu.CompilerParams(use_tc_tiling_on_sc=False),
        out_type=jax.ShapeDtypeStruct((_NNP, NFILT), jnp.float32),
        scratch_types=[
            pltpu.VMEM((_PAIRS, 2 * SEQ), jnp.int32),
            pltpu.VMEM((_CH * SEQ, NFILT), jnp.float32),
            pltpu.VMEM((_CH, NFILT), jnp.float32),
            pltpu.SemaphoreType.DMA,
        ],
    )(_enc_body)
    return k(table, idx_pairs)


_BM = 400
_NBLK = V // _BM


def _adj_body(adj_ref, sup_ref, b0_ref, h_ref, s_ref):
    i = pl.program_id(0)
    blk = adj_ref[:, :]
    h_ref[:, :] = jnp.maximum(
        jnp.dot(blk, sup_ref[:, :], preferred_element_type=jnp.float32)
        + b0_ref[:, :],
        0.0,
    )
    csum = jnp.sum(blk, axis=0, keepdims=True)

    @pl.when(i == 0)
    def _():
        s_ref[:, :] = csum

    @pl.when(i > 0)
    def _():
        s_ref[:, :] = s_ref[:, :] + csum


def _adj_pass(adj, support, b0):
    return pl.pallas_call(
        _adj_body,
        grid=(_NBLK,),
        in_specs=[
            pl.BlockSpec((_BM, V), lambda i: (i, 0)),
            pl.BlockSpec((V, NFILT), lambda i: (0, 0)),
            pl.BlockSpec((1, NFILT), lambda i: (0, 0)),
        ],
        out_specs=[
            pl.BlockSpec((_BM, NFILT), lambda i: (i, 0)),
            pl.BlockSpec((1, V), lambda i: (0, 0)),
        ],
        out_shape=[
            jax.ShapeDtypeStruct((V, NFILT), jnp.float32),
            jax.ShapeDtypeStruct((1, V), jnp.float32),
        ],
    )(adj, support, b0)


def _support_body(el_ref, er_ref, vf_ref, w2v_ref, vfw_ref, out_ref):
    el = el_ref[:, :]
    er = er_ref[:, :]
    xsi = jnp.concatenate([el * er, jnp.abs(el - er)], axis=1)
    sup_s = jnp.dot(xsi, w2v_ref[:, :], preferred_element_type=jnp.float32)
    sup_v = jnp.dot(vf_ref[:, :], vfw_ref[:, :], preferred_element_type=jnp.float32)
    out_ref[:, :] = jnp.concatenate([sup_s, sup_v], axis=1)


def _support(encl, encr, v_feature, w2v_w0, vf_w0):
    return pl.pallas_call(
        _support_body,
        out_shape=jax.ShapeDtypeStruct((V, NFILT), jnp.float32),
    )(encl, encr, v_feature, w2v_w0, vf_w0)


def _head_body(s_ref, h_ref, w2v1_ref, b2v1_ref, vf1_ref, bvf1_ref, g_ref,
               rw0_ref, rb0_ref, rw1_ref, rb1_ref, out_ref):
    z = jnp.dot(s_ref[:, :], h_ref[:, :], preferred_element_type=jnp.float32) / V
    zs = jnp.dot(z[:, :16], w2v1_ref[:, :], preferred_element_type=jnp.float32) + b2v1_ref[:, :]
    zv = jnp.dot(z[:, 16:], vf1_ref[:, :], preferred_element_type=jnp.float32) + bvf1_ref[:, :]
    x = jnp.concatenate([zs, zv, g_ref[:, :]], axis=1)
    hh = jnp.maximum(
        jnp.dot(x, rw0_ref[:, :], preferred_element_type=jnp.float32) + rb0_ref[:, :],
        0.0,
    )
    pre = jnp.dot(hh, rw1_ref[:, :], preferred_element_type=jnp.float32) + rb1_ref[:, :]
    out_ref[:, :] = jax.nn.sigmoid(pre)


def _head(s, h, w2v_w1, w2v_b1, vf_w1, vf_b1, g_feature,
          reg_w0, reg_b0, reg_w1, reg_b1):
    return pl.pallas_call(
        _head_body,
        out_shape=jax.ShapeDtypeStruct((1, 1), jnp.float32),
    )(s, h, w2v_w1, w2v_b1.reshape(1, -1), vf_w1, vf_b1.reshape(1, -1),
      g_feature.reshape(1, -1), reg_w0, reg_b0.reshape(1, -1),
      reg_w1, reg_b1.reshape(1, -1))


def kernel(x_l, x_r, v_feature, adj, g_feature, emb_table, conv_w, conv_b,
           w2v_w0, w2v_b0, w2v_w1, w2v_b1, vf_w0, vf_b0, vf_w1, vf_b1,
           reg_w0, reg_b0, reg_w1, reg_b1):
    table = _project_table(emb_table, conv_w, conv_b)

    idx = jnp.concatenate([x_l, x_r], axis=0)
    idx = jnp.pad(idx, ((0, _NNP - _NN), (0, 0)))
    idx_pairs = idx.reshape(_NNP // 2, 2 * SEQ)

    enc = _encode_sc(table, idx_pairs)
    encl = enc[:V]
    encr = enc[V:_NN]

    support = _support(encl, encr, v_feature, w2v_w0, vf_w0)
    b0 = jnp.concatenate([w2v_b0, vf_b0]).reshape(1, NFILT)
    h, s = _adj_pass(adj, support, b0)

    out = _head(s, h, w2v_w1, w2v_b1, vf_w1, vf_b1, g_feature,
                reg_w0, reg_b0, reg_w1, reg_b1)
    return out.reshape(1)


# speedup vs baseline: 5.2364x; 1.0091x over previous
import functools

import jax
import jax.numpy as jnp
from jax import lax
from jax.experimental import pallas as pl
from jax.experimental.pallas import tpu as pltpu
from jax.experimental.pallas import tpu_sc as plsc

V = 10000
SEQ = 50
VOCAB = 30000
NFILT = 32

_NC = 2
_NS = 16
_NW = _NC * _NS
_NN = 2 * V
_NNP = 20480
_PW = _NNP // _NW
_CH = 32
_NCHUNK = _PW // _CH
_PAIRS = _CH // 2


def _proj_body(emb_ref, cwt_ref, cb_ref, out_ref):
    out_ref[:, :] = (
        jnp.dot(emb_ref[:, :], cwt_ref[:, :], preferred_element_type=jnp.float32)
        + cb_ref[:, :]
    )


def _project_table(emb_table, conv_w, conv_b):
    return pl.pallas_call(
        _proj_body,
        out_shape=jax.ShapeDtypeStruct((VOCAB, NFILT), jnp.float32),
    )(emb_table, conv_w.T, conv_b.reshape(1, NFILT))


def _enc_body(table_hbm, idx_hbm, out_hbm, idx_v, rows_v, out_v, sem):
    wid = lax.axis_index("s") * _NC + lax.axis_index("c")
    flat_base = wid * _PW * SEQ
    node_base = wid * _PW

    def chunk_body(g, carry):
        pltpu.sync_copy(idx_hbm.at[pl.ds(flat_base + g * _CH * SEQ, _CH * SEQ)], idx_v)
        pltpu.async_copy(table_hbm.at[idx_v], rows_v, sem).wait()

        def node_body(n, c2):
            acc0 = jnp.zeros((16,), jnp.float32)
            acc1 = jnp.zeros((16,), jnp.float32)
            for s in range(SEQ):
                r = n * SEQ + s
                acc0 = jnp.maximum(acc0, rows_v[r, pl.ds(0, 16)])
                acc1 = jnp.maximum(acc1, rows_v[r, pl.ds(16, 16)])
            out_v[n, pl.ds(0, 16)] = acc0
            out_v[n, pl.ds(16, 16)] = acc1
            return c2

        lax.fori_loop(0, _CH, node_body, 0)
        pltpu.sync_copy(out_v, out_hbm.at[pl.ds(node_base + g * _CH, _CH)])
        return carry

    lax.fori_loop(0, _NCHUNK, chunk_body, 0)


def _encode_sc(table, idx_flat):
    mesh = plsc.VectorSubcoreMesh(core_axis_name="c", subcore_axis_name="s")
    k = functools.partial(
        pl.kernel,
        mesh=mesh,
        compiler_params=pltpu.CompilerParams(use_tc_tiling_on_sc=False),
        out_type=jax.ShapeDtypeStruct((_NNP, NFILT), jnp.float32),
        scratch_types=[
            pltpu.VMEM((_CH * SEQ,), jnp.int32),
            pltpu.VMEM((_CH * SEQ, NFILT), jnp.float32),
            pltpu.VMEM((_CH, NFILT), jnp.float32),
            pltpu.SemaphoreType.DMA,
        ],
    )(_enc_body)
    return k(table, idx_flat)


_BM = 400
_NBLK = V // _BM


def _adj_body(adj_ref, sup_ref, b0_ref, h_ref, s_ref):
    i = pl.program_id(0)
    blk = adj_ref[:, :]
    h_ref[:, :] = jnp.maximum(
        jnp.dot(blk, sup_ref[:, :], preferred_element_type=jnp.float32)
        + b0_ref[:, :],
        0.0,
    )
    csum = jnp.sum(blk, axis=0, keepdims=True)

    @pl.when(i == 0)
    def _():
        s_ref[:, :] = csum

    @pl.when(i > 0)
    def _():
        s_ref[:, :] = s_ref[:, :] + csum


def _adj_pass(adj, support, b0):
    return pl.pallas_call(
        _adj_body,
        grid=(_NBLK,),
        in_specs=[
            pl.BlockSpec((_BM, V), lambda i: (i, 0)),
            pl.BlockSpec((V, NFILT), lambda i: (0, 0)),
            pl.BlockSpec((1, NFILT), lambda i: (0, 0)),
        ],
        out_specs=[
            pl.BlockSpec((_BM, NFILT), lambda i: (i, 0)),
            pl.BlockSpec((1, V), lambda i: (0, 0)),
        ],
        out_shape=[
            jax.ShapeDtypeStruct((V, NFILT), jnp.float32),
            jax.ShapeDtypeStruct((1, V), jnp.float32),
        ],
    )(adj, support, b0)


def _support_body(el_ref, er_ref, vf_ref, w2v_ref, vfw_ref, out_ref):
    el = el_ref[:, :]
    er = er_ref[:, :]
    xsi = jnp.concatenate([el * er, jnp.abs(el - er)], axis=1)
    sup_s = jnp.dot(xsi, w2v_ref[:, :], preferred_element_type=jnp.float32)
    sup_v = jnp.dot(vf_ref[:, :], vfw_ref[:, :], preferred_element_type=jnp.float32)
    out_ref[:, :] = jnp.concatenate([sup_s, sup_v], axis=1)


def _support(encl, encr, v_feature, w2v_w0, vf_w0):
    return pl.pallas_call(
        _support_body,
        out_shape=jax.ShapeDtypeStruct((V, NFILT), jnp.float32),
    )(encl, encr, v_feature, w2v_w0, vf_w0)


def _head_body(s_ref, h_ref, w2v1_ref, b2v1_ref, vf1_ref, bvf1_ref, g_ref,
               rw0_ref, rb0_ref, rw1_ref, rb1_ref, out_ref):
    z = jnp.dot(s_ref[:, :], h_ref[:, :], preferred_element_type=jnp.float32) / V
    zs = jnp.dot(z[:, :16], w2v1_ref[:, :], preferred_element_type=jnp.float32) + b2v1_ref[:, :]
    zv = jnp.dot(z[:, 16:], vf1_ref[:, :], preferred_element_type=jnp.float32) + bvf1_ref[:, :]
    x = jnp.concatenate([zs, zv, g_ref[:, :]], axis=1)
    hh = jnp.maximum(
        jnp.dot(x, rw0_ref[:, :], preferred_element_type=jnp.float32) + rb0_ref[:, :],
        0.0,
    )
    pre = jnp.dot(hh, rw1_ref[:, :], preferred_element_type=jnp.float32) + rb1_ref[:, :]
    out_ref[:, :] = jax.nn.sigmoid(pre)


def _head(s, h, w2v_w1, w2v_b1, vf_w1, vf_b1, g_feature,
          reg_w0, reg_b0, reg_w1, reg_b1):
    return pl.pallas_call(
        _head_body,
        out_shape=jax.ShapeDtypeStruct((1, 1), jnp.float32),
    )(s, h, w2v_w1, w2v_b1.reshape(1, -1), vf_w1, vf_b1.reshape(1, -1),
      g_feature.reshape(1, -1), reg_w0, reg_b0.reshape(1, -1),
      reg_w1, reg_b1.reshape(1, -1))


def kernel(x_l, x_r, v_feature, adj, g_feature, emb_table, conv_w, conv_b,
           w2v_w0, w2v_b0, w2v_w1, w2v_b1, vf_w0, vf_b0, vf_w1, vf_b1,
           reg_w0, reg_b0, reg_w1, reg_b1):
    table = _project_table(emb_table, conv_w, conv_b)

    idx = jnp.concatenate([x_l, x_r], axis=0)
    idx = jnp.pad(idx, ((0, _NNP - _NN), (0, 0)))
    idx_flat = idx.reshape(_NNP * SEQ)

    enc = _encode_sc(table, idx_flat)
    encl = enc[:V]
    encr = enc[V:_NN]

    support = _support(encl, encr, v_feature, w2v_w0, vf_w0)
    b0 = jnp.concatenate([w2v_b0, vf_b0]).reshape(1, NFILT)
    h, s = _adj_pass(adj, support, b0)

    out = _head(s, h, w2v_w1, w2v_b1, vf_w1, vf_b1, g_feature,
                reg_w0, reg_b0, reg_w1, reg_b1)
    return out.reshape(1)


# speedup vs baseline: 7.4747x; 1.4275x over previous
import functools

import jax
import jax.numpy as jnp
from jax import lax
from jax.experimental import pallas as pl
from jax.experimental.pallas import tpu as pltpu
from jax.experimental.pallas import tpu_sc as plsc

V = 10000
SEQ = 50
VOCAB = 30000
NFILT = 32

_NC = 2
_NS = 16
_NW = _NC * _NS
_NN = 2 * V
_NNP = 20480
_PW = _NNP // _NW
_CH = 32
_NCHUNK = _PW // _CH
_PAIRS = _CH // 2


def _proj_body(emb_ref, cwt_ref, cb_ref, out_ref):
    out_ref[:, :] = (
        jnp.dot(emb_ref[:, :], cwt_ref[:, :], preferred_element_type=jnp.float32)
        + cb_ref[:, :]
    ).astype(jnp.bfloat16)


def _project_table(emb_table, conv_w, conv_b):
    return pl.pallas_call(
        _proj_body,
        out_shape=jax.ShapeDtypeStruct((VOCAB, NFILT), jnp.bfloat16),
    )(emb_table, conv_w.T, conv_b.reshape(1, NFILT))


def _enc_body(table_hbm, idx_hbm, out_hbm, idx_v, rows_v, out_v, sem):
    wid = lax.axis_index("s") * _NC + lax.axis_index("c")
    flat_base = wid * _PW * SEQ
    node_base = wid * _PW

    def chunk_body(g, carry):
        pltpu.sync_copy(idx_hbm.at[pl.ds(flat_base + g * _CH * SEQ, _CH * SEQ)], idx_v)
        pltpu.async_copy(table_hbm.at[idx_v], rows_v, sem).wait()

        def node_body(n, c2):
            acc = jnp.zeros((2 * 16,), jnp.bfloat16)
            for s in range(SEQ):
                acc = jnp.maximum(acc, rows_v[n * SEQ + s, :])
            out_v[n, :] = acc
            return c2

        lax.fori_loop(0, _CH, node_body, 0)
        pltpu.sync_copy(out_v, out_hbm.at[pl.ds(node_base + g * _CH, _CH)])
        return carry

    lax.fori_loop(0, _NCHUNK, chunk_body, 0)


def _encode_sc(table, idx_flat):
    mesh = plsc.VectorSubcoreMesh(core_axis_name="c", subcore_axis_name="s")
    k = functools.partial(
        pl.kernel,
        mesh=mesh,
        compiler_params=pltpu.CompilerParams(use_tc_tiling_on_sc=False),
        out_type=jax.ShapeDtypeStruct((_NNP, NFILT), jnp.bfloat16),
        scratch_types=[
            pltpu.VMEM((_CH * SEQ,), jnp.int32),
            pltpu.VMEM((_CH * SEQ, NFILT), jnp.bfloat16),
            pltpu.VMEM((_CH, NFILT), jnp.bfloat16),
            pltpu.SemaphoreType.DMA,
        ],
    )(_enc_body)
    return k(table, idx_flat)


_BM = 400
_NBLK = V // _BM


def _adj_body(adj_ref, sup_ref, b0_ref, h_ref, s_ref):
    i = pl.program_id(0)
    blk = adj_ref[:, :]
    h_ref[:, :] = jnp.maximum(
        jnp.dot(blk, sup_ref[:, :], preferred_element_type=jnp.float32)
        + b0_ref[:, :],
        0.0,
    )
    csum = jnp.sum(blk, axis=0, keepdims=True)

    @pl.when(i == 0)
    def _():
        s_ref[:, :] = csum

    @pl.when(i > 0)
    def _():
        s_ref[:, :] = s_ref[:, :] + csum


def _adj_pass(adj, support, b0):
    return pl.pallas_call(
        _adj_body,
        grid=(_NBLK,),
        in_specs=[
            pl.BlockSpec((_BM, V), lambda i: (i, 0)),
            pl.BlockSpec((V, NFILT), lambda i: (0, 0)),
            pl.BlockSpec((1, NFILT), lambda i: (0, 0)),
        ],
        out_specs=[
            pl.BlockSpec((_BM, NFILT), lambda i: (i, 0)),
            pl.BlockSpec((1, V), lambda i: (0, 0)),
        ],
        out_shape=[
            jax.ShapeDtypeStruct((V, NFILT), jnp.float32),
            jax.ShapeDtypeStruct((1, V), jnp.float32),
        ],
    )(adj, support, b0)


def _support_body(el_ref, er_ref, vf_ref, w2v_ref, vfw_ref, out_ref):
    el = el_ref[:, :].astype(jnp.float32)
    er = er_ref[:, :].astype(jnp.float32)
    xsi = jnp.concatenate([el * er, jnp.abs(el - er)], axis=1)
    sup_s = jnp.dot(xsi, w2v_ref[:, :], preferred_element_type=jnp.float32)
    sup_v = jnp.dot(vf_ref[:, :], vfw_ref[:, :], preferred_element_type=jnp.float32)
    out_ref[:, :] = jnp.concatenate([sup_s, sup_v], axis=1)


def _support(encl, encr, v_feature, w2v_w0, vf_w0):
    return pl.pallas_call(
        _support_body,
        out_shape=jax.ShapeDtypeStruct((V, NFILT), jnp.float32),
    )(encl, encr, v_feature, w2v_w0, vf_w0)


def _head_body(s_ref, h_ref, w2v1_ref, b2v1_ref, vf1_ref, bvf1_ref, g_ref,
               rw0_ref, rb0_ref, rw1_ref, rb1_ref, out_ref):
    z = jnp.dot(s_ref[:, :], h_ref[:, :], preferred_element_type=jnp.float32) / V
    zs = jnp.dot(z[:, :16], w2v1_ref[:, :], preferred_element_type=jnp.float32) + b2v1_ref[:, :]
    zv = jnp.dot(z[:, 16:], vf1_ref[:, :], preferred_element_type=jnp.float32) + bvf1_ref[:, :]
    x = jnp.concatenate([zs, zv, g_ref[:, :]], axis=1)
    hh = jnp.maximum(
        jnp.dot(x, rw0_ref[:, :], preferred_element_type=jnp.float32) + rb0_ref[:, :],
        0.0,
    )
    pre = jnp.dot(hh, rw1_ref[:, :], preferred_element_type=jnp.float32) + rb1_ref[:, :]
    out_ref[:, :] = jax.nn.sigmoid(pre)


def _head(s, h, w2v_w1, w2v_b1, vf_w1, vf_b1, g_feature,
          reg_w0, reg_b0, reg_w1, reg_b1):
    return pl.pallas_call(
        _head_body,
        out_shape=jax.ShapeDtypeStruct((1, 1), jnp.float32),
    )(s, h, w2v_w1, w2v_b1.reshape(1, -1), vf_w1, vf_b1.reshape(1, -1),
      g_feature.reshape(1, -1), reg_w0, reg_b0.reshape(1, -1),
      reg_w1, reg_b1.reshape(1, -1))


def kernel(x_l, x_r, v_feature, adj, g_feature, emb_table, conv_w, conv_b,
           w2v_w0, w2v_b0, w2v_w1, w2v_b1, vf_w0, vf_b0, vf_w1, vf_b1,
           reg_w0, reg_b0, reg_w1, reg_b1):
    table = _project_table(emb_table, conv_w, conv_b)

    idx = jnp.concatenate([x_l, x_r], axis=0)
    idx = jnp.pad(idx, ((0, _NNP - _NN), (0, 0)))
    idx_flat = idx.reshape(_NNP * SEQ)

    enc = _encode_sc(table, idx_flat)
    encl = enc[:V]
    encr = enc[V:_NN]

    support = _support(encl, encr, v_feature, w2v_w0, vf_w0)
    b0 = jnp.concatenate([w2v_b0, vf_b0]).reshape(1, NFILT)
    h, s = _adj_pass(adj, support, b0)

    out = _head(s, h, w2v_w1, w2v_b1, vf_w1, vf_b1, g_feature,
                reg_w0, reg_b0, reg_w1, reg_b1)
    return out.reshape(1)


# speedup vs baseline: 7.8770x; 1.0538x over previous
import functools

import jax
import jax.numpy as jnp
from jax import lax
from jax.experimental import pallas as pl
from jax.experimental.pallas import tpu as pltpu
from jax.experimental.pallas import tpu_sc as plsc

V = 10000
SEQ = 50
VOCAB = 30000
NFILT = 32

_NC = 2
_NS = 16
_NW = _NC * _NS
_NN = 2 * V
_NNP = 20480
_PW = _NNP // _NW
_CH = 32
_NCHUNK = _PW // _CH
_PAIRS = _CH // 2


def _proj_body(emb_ref, cwt_ref, cb_ref, out_ref):
    out_ref[:, :] = (
        jnp.dot(emb_ref[:, :], cwt_ref[:, :], preferred_element_type=jnp.float32)
        + cb_ref[:, :]
    ).astype(jnp.bfloat16)


def _project_table(emb_table, conv_w, conv_b):
    return pl.pallas_call(
        _proj_body,
        out_shape=jax.ShapeDtypeStruct((VOCAB, NFILT), jnp.bfloat16),
    )(emb_table, conv_w.T, conv_b.reshape(1, NFILT))


_CHROWS = _CH * SEQ


def _enc_body(table_hbm, idx_hbm, out_hbm, idx_all, rows_v, out_v, sem0, sem1):
    wid = lax.axis_index("s") * _NC + lax.axis_index("c")
    sems = (sem0, sem1)

    pltpu.sync_copy(idx_hbm.at[pl.ds(wid * _PW * SEQ, _PW * SEQ)], idx_all)

    def fire(g, buf):
        pltpu.async_copy(
            table_hbm.at[idx_all.at[pl.ds(g * _CHROWS, _CHROWS)]],
            rows_v.at[buf],
            sems[buf],
        )

    def drain(buf):
        pltpu.make_async_copy(
            table_hbm.at[idx_all.at[pl.ds(0, _CHROWS)]],
            rows_v.at[buf],
            sems[buf],
        ).wait()

    def compute(g, buf):
        def node_body(n, c2):
            acc = jnp.zeros((2 * 16,), jnp.bfloat16)
            for s in range(SEQ):
                acc = jnp.maximum(acc, rows_v[buf, n * SEQ + s, :])
            out_v[g * _CH + n, :] = acc
            return c2

        lax.fori_loop(0, _CH, node_body, 0)

    fire(0, 0)

    def pair_body(k, carry):
        g0 = 2 * k
        fire(g0 + 1, 1)
        drain(0)
        compute(g0, 0)

        @pl.when(k < _NCHUNK // 2 - 1)
        def _():
            fire(g0 + 2, 0)

        drain(1)
        compute(g0 + 1, 1)
        return carry

    lax.fori_loop(0, _NCHUNK // 2, pair_body, 0)
    pltpu.sync_copy(out_v, out_hbm.at[pl.ds(wid * _PW, _PW)])


def _encode_sc(table, idx_flat):
    mesh = plsc.VectorSubcoreMesh(core_axis_name="c", subcore_axis_name="s")
    k = functools.partial(
        pl.kernel,
        mesh=mesh,
        compiler_params=pltpu.CompilerParams(use_tc_tiling_on_sc=False),
        out_type=jax.ShapeDtypeStruct((_NNP, NFILT), jnp.bfloat16),
        scratch_types=[
            pltpu.VMEM((_PW * SEQ,), jnp.int32),
            pltpu.VMEM((2, _CHROWS, NFILT), jnp.bfloat16),
            pltpu.VMEM((_PW, NFILT), jnp.bfloat16),
            pltpu.SemaphoreType.DMA,
            pltpu.SemaphoreType.DMA,
        ],
    )(_enc_body)
    return k(table, idx_flat)


_BM = 400
_NBLK = V // _BM


def _adj_body(adj_ref, sup_ref, b0_ref, h_ref, s_ref):
    i = pl.program_id(0)
    blk = adj_ref[:, :]
    h_ref[:, :] = jnp.maximum(
        jnp.dot(blk, sup_ref[:, :], preferred_element_type=jnp.float32)
        + b0_ref[:, :],
        0.0,
    )
    csum = jnp.sum(blk, axis=0, keepdims=True)

    @pl.when(i == 0)
    def _():
        s_ref[:, :] = csum

    @pl.when(i > 0)
    def _():
        s_ref[:, :] = s_ref[:, :] + csum


def _adj_pass(adj, support, b0):
    return pl.pallas_call(
        _adj_body,
        grid=(_NBLK,),
        in_specs=[
            pl.BlockSpec((_BM, V), lambda i: (i, 0)),
            pl.BlockSpec((V, NFILT), lambda i: (0, 0)),
            pl.BlockSpec((1, NFILT), lambda i: (0, 0)),
        ],
        out_specs=[
            pl.BlockSpec((_BM, NFILT), lambda i: (i, 0)),
            pl.BlockSpec((1, V), lambda i: (0, 0)),
        ],
        out_shape=[
            jax.ShapeDtypeStruct((V, NFILT), jnp.float32),
            jax.ShapeDtypeStruct((1, V), jnp.float32),
        ],
    )(adj, support, b0)


def _support_body(el_ref, er_ref, vf_ref, w2v_ref, vfw_ref, out_ref):
    el = el_ref[:, :].astype(jnp.float32)
    er = er_ref[:, :].astype(jnp.float32)
    xsi = jnp.concatenate([el * er, jnp.abs(el - er)], axis=1)
    sup_s = jnp.dot(xsi, w2v_ref[:, :], preferred_element_type=jnp.float32)
    sup_v = jnp.dot(vf_ref[:, :], vfw_ref[:, :], preferred_element_type=jnp.float32)
    out_ref[:, :] = jnp.concatenate([sup_s, sup_v], axis=1)


def _support(encl, encr, v_feature, w2v_w0, vf_w0):
    return pl.pallas_call(
        _support_body,
        out_shape=jax.ShapeDtypeStruct((V, NFILT), jnp.float32),
    )(encl, encr, v_feature, w2v_w0, vf_w0)


def _head_body(s_ref, h_ref, w2v1_ref, b2v1_ref, vf1_ref, bvf1_ref, g_ref,
               rw0_ref, rb0_ref, rw1_ref, rb1_ref, out_ref):
    z = jnp.dot(s_ref[:, :], h_ref[:, :], preferred_element_type=jnp.float32) / V
    zs = jnp.dot(z[:, :16], w2v1_ref[:, :], preferred_element_type=jnp.float32) + b2v1_ref[:, :]
    zv = jnp.dot(z[:, 16:], vf1_ref[:, :], preferred_element_type=jnp.float32) + bvf1_ref[:, :]
    x = jnp.concatenate([zs, zv, g_ref[:, :]], axis=1)
    hh = jnp.maximum(
        jnp.dot(x, rw0_ref[:, :], preferred_element_type=jnp.float32) + rb0_ref[:, :],
        0.0,
    )
    pre = jnp.dot(hh, rw1_ref[:, :], preferred_element_type=jnp.float32) + rb1_ref[:, :]
    out_ref[:, :] = jax.nn.sigmoid(pre)


def _head(s, h, w2v_w1, w2v_b1, vf_w1, vf_b1, g_feature,
          reg_w0, reg_b0, reg_w1, reg_b1):
    return pl.pallas_call(
        _head_body,
        out_shape=jax.ShapeDtypeStruct((1, 1), jnp.float32),
    )(s, h, w2v_w1, w2v_b1.reshape(1, -1), vf_w1, vf_b1.reshape(1, -1),
      g_feature.reshape(1, -1), reg_w0, reg_b0.reshape(1, -1),
      reg_w1, reg_b1.reshape(1, -1))


def kernel(x_l, x_r, v_feature, adj, g_feature, emb_table, conv_w, conv_b,
           w2v_w0, w2v_b0, w2v_w1, w2v_b1, vf_w0, vf_b0, vf_w1, vf_b1,
           reg_w0, reg_b0, reg_w1, reg_b1):
    table = _project_table(emb_table, conv_w, conv_b)

    idx = jnp.concatenate([x_l, x_r], axis=0)
    idx = jnp.pad(idx, ((0, _NNP - _NN), (0, 0)))
    idx_flat = idx.reshape(_NNP * SEQ)

    enc = _encode_sc(table, idx_flat)
    encl = enc[:V]
    encr = enc[V:_NN]

    support = _support(encl, encr, v_feature, w2v_w0, vf_w0)
    b0 = jnp.concatenate([w2v_b0, vf_b0]).reshape(1, NFILT)
    h, s = _adj_pass(adj, support, b0)

    out = _head(s, h, w2v_w1, w2v_b1, vf_w1, vf_b1, g_feature,
                reg_w0, reg_b0, reg_w1, reg_b1)
    return out.reshape(1)


# speedup vs baseline: 11.9945x; 1.5227x over previous
import functools

import jax
import jax.numpy as jnp
from jax import lax
from jax.experimental import pallas as pl
from jax.experimental.pallas import tpu as pltpu
from jax.experimental.pallas import tpu_sc as plsc

V = 10000
SEQ = 50
VOCAB = 30000
NFILT = 32

_NC = 2
_NS = 16
_NW = _NC * _NS
_NN = 2 * V
_NNP = 20480
_PW = _NNP // _NW
_CH = 32
_NCHUNK = _PW // _CH
_CHROWS = _CH * SEQ


def _proj_body(emb_ref, cwt_ref, cb_ref, out_ref):
    out_ref[:, :] = (
        jnp.dot(emb_ref[:, :], cwt_ref[:, :], preferred_element_type=jnp.float32)
        + cb_ref[:, :]
    ).astype(jnp.bfloat16)


def _project_table(emb_table, conv_w, conv_b):
    return pl.pallas_call(
        _proj_body,
        out_shape=jax.ShapeDtypeStruct((VOCAB, NFILT), jnp.bfloat16),
    )(emb_table, conv_w.T, conv_b.reshape(1, NFILT))


_TPS = VOCAB // _NS


def _enc_body(table_hbm, idx_hbm, out_hbm, tab_sh, idx_all, rows_v, out_v,
              sem0, sem1):
    sid = lax.axis_index("s")
    wid = sid * _NC + lax.axis_index("c")
    sems = (sem0, sem1)

    pltpu.sync_copy(table_hbm.at[pl.ds(sid * _TPS, _TPS)],
                    tab_sh.at[pl.ds(sid * _TPS, _TPS)])
    plsc.subcore_barrier()

    pltpu.sync_copy(idx_hbm.at[pl.ds(wid * _PW * SEQ, _PW * SEQ)], idx_all)

    def fire(g, buf):
        pltpu.async_copy(
            tab_sh.at[idx_all.at[pl.ds(g * _CHROWS, _CHROWS)]],
            rows_v.at[buf],
            sems[buf],
        )

    def drain(buf):
        pltpu.make_async_copy(
            tab_sh.at[idx_all.at[pl.ds(0, _CHROWS)]],
            rows_v.at[buf],
            sems[buf],
        ).wait()

    def compute(g, buf):
        def node_body(n, c2):
            acc = jnp.zeros((2 * 16,), jnp.bfloat16)
            for s in range(SEQ):
                acc = jnp.maximum(acc, rows_v[buf, n * SEQ + s, :])
            out_v[g * _CH + n, :] = acc
            return c2

        lax.fori_loop(0, _CH, node_body, 0)

    fire(0, 0)

    def pair_body(k, carry):
        g0 = 2 * k
        fire(g0 + 1, 1)
        drain(0)
        compute(g0, 0)

        @pl.when(k < _NCHUNK // 2 - 1)
        def _():
            fire(g0 + 2, 0)

        drain(1)
        compute(g0 + 1, 1)
        return carry

    lax.fori_loop(0, _NCHUNK // 2, pair_body, 0)
    pltpu.sync_copy(out_v, out_hbm.at[pl.ds(wid * _PW, _PW)])


def _encode_sc(table, idx_flat):
    mesh = plsc.VectorSubcoreMesh(core_axis_name="c", subcore_axis_name="s")
    k = functools.partial(
        pl.kernel,
        mesh=mesh,
        compiler_params=pltpu.CompilerParams(use_tc_tiling_on_sc=False),
        out_type=jax.ShapeDtypeStruct((_NNP, NFILT), jnp.bfloat16),
        scratch_types=[
            pltpu.VMEM_SHARED((VOCAB, NFILT), jnp.bfloat16),
            pltpu.VMEM((_PW * SEQ,), jnp.int32),
            pltpu.VMEM((2, _CHROWS, NFILT), jnp.bfloat16),
            pltpu.VMEM((_PW, NFILT), jnp.bfloat16),
            pltpu.SemaphoreType.DMA,
            pltpu.SemaphoreType.DMA,
        ],
    )(_enc_body)
    return k(table, idx_flat)


_BM = 400
_NBLK = V // _BM


def _support_body(el_ref, er_ref, vf_ref, w2v_ref, vfw_ref, out_ref):
    el = el_ref[:, :].astype(jnp.float32)
    er = er_ref[:, :].astype(jnp.float32)
    xsi = jnp.concatenate([el * er, jnp.abs(el - er)], axis=1)
    sup_s = jnp.dot(xsi, w2v_ref[:, :], preferred_element_type=jnp.float32)
    sup_v = jnp.dot(vf_ref[:, :], vfw_ref[:, :], preferred_element_type=jnp.float32)
    out_ref[:, :] = jnp.concatenate([sup_s, sup_v], axis=1)


def _support(encl, encr, v_feature, w2v_w0, vf_w0):
    return pl.pallas_call(
        _support_body,
        out_shape=jax.ShapeDtypeStruct((V, NFILT), jnp.float32),
    )(encl, encr, v_feature, w2v_w0, vf_w0)


def _adj_body(adj_ref, sup_ref, b0_ref, h_ref, s_ref):
    i = pl.program_id(0)
    blk = adj_ref[:, :]
    h_ref[:, :] = jnp.maximum(
        jnp.dot(blk, sup_ref[:, :], preferred_element_type=jnp.float32)
        + b0_ref[:, :],
        0.0,
    )
    csum = jnp.sum(blk, axis=0, keepdims=True)

    @pl.when(i == 0)
    def _():
        s_ref[:, :] = csum

    @pl.when(i > 0)
    def _():
        s_ref[:, :] = s_ref[:, :] + csum


def _adj_pass(adj, support, b0):
    return pl.pallas_call(
        _adj_body,
        grid=(_NBLK,),
        in_specs=[
            pl.BlockSpec((_BM, V), lambda i: (i, 0)),
            pl.BlockSpec((V, NFILT), lambda i: (0, 0)),
            pl.BlockSpec((1, NFILT), lambda i: (0, 0)),
        ],
        out_specs=[
            pl.BlockSpec((_BM, NFILT), lambda i: (i, 0)),
            pl.BlockSpec((1, V), lambda i: (0, 0)),
        ],
        out_shape=[
            jax.ShapeDtypeStruct((V, NFILT), jnp.float32),
            jax.ShapeDtypeStruct((1, V), jnp.float32),
        ],
    )(adj, support, b0)


def _head_body(s_ref, h_ref, w2v1_ref, b2v1_ref, vf1_ref, bvf1_ref, g_ref,
               rw0_ref, rb0_ref, rw1_ref, rb1_ref, out_ref):
    z = jnp.dot(s_ref[:, :], h_ref[:, :], preferred_element_type=jnp.float32) / V
    zs = jnp.dot(z[:, :16], w2v1_ref[:, :], preferred_element_type=jnp.float32) + b2v1_ref[:, :]
    zv = jnp.dot(z[:, 16:], vf1_ref[:, :], preferred_element_type=jnp.float32) + bvf1_ref[:, :]
    x = jnp.concatenate([zs, zv, g_ref[:, :]], axis=1)
    hh = jnp.maximum(
        jnp.dot(x, rw0_ref[:, :], preferred_element_type=jnp.float32) + rb0_ref[:, :],
        0.0,
    )
    pre = jnp.dot(hh, rw1_ref[:, :], preferred_element_type=jnp.float32) + rb1_ref[:, :]
    out_ref[:, :] = jax.nn.sigmoid(pre)


def _head(s, h, w2v_w1, w2v_b1, vf_w1, vf_b1, g_feature,
          reg_w0, reg_b0, reg_w1, reg_b1):
    return pl.pallas_call(
        _head_body,
        out_shape=jax.ShapeDtypeStruct((1, 1), jnp.float32),
    )(s, h, w2v_w1, w2v_b1.reshape(1, -1), vf_w1, vf_b1.reshape(1, -1),
      g_feature.reshape(1, -1), reg_w0, reg_b0.reshape(1, -1),
      reg_w1, reg_b1.reshape(1, -1))


def kernel(x_l, x_r, v_feature, adj, g_feature, emb_table, conv_w, conv_b,
           w2v_w0, w2v_b0, w2v_w1, w2v_b1, vf_w0, vf_b0, vf_w1, vf_b1,
           reg_w0, reg_b0, reg_w1, reg_b1):
    table = _project_table(emb_table, conv_w, conv_b)

    idx = jnp.concatenate([x_l, x_r], axis=0)
    idx = jnp.pad(idx, ((0, _NNP - _NN), (0, 0)))
    idx_flat = idx.reshape(_NNP * SEQ)

    enc = _encode_sc(table, idx_flat)

    support = _support(enc[:V], enc[V:_NN], v_feature, w2v_w0, vf_w0)
    b0 = jnp.concatenate([w2v_b0, vf_b0]).reshape(1, NFILT)
    h, s = _adj_pass(adj, support, b0)

    out = _head(s, h, w2v_w1, w2v_b1, vf_w1, vf_b1, g_feature,
                reg_w0, reg_b0, reg_w1, reg_b1)
    return out.reshape(1)


# speedup vs baseline: 12.3357x; 1.0284x over previous
import functools

import jax
import jax.numpy as jnp
from jax import lax
from jax.experimental import pallas as pl
from jax.experimental.pallas import tpu as pltpu
from jax.experimental.pallas import tpu_sc as plsc

V = 10000
SEQ = 50
VOCAB = 30000
NFILT = 32
NFEAT_G = 32

_NC = 2
_NS = 16
_NW = _NC * _NS
_NN = 2 * V
_NNP = 20480
_PW = _NNP // _NW
_CH = 32
_NCHUNK = _PW // _CH
_CHROWS = _CH * SEQ


def _proj_body(emb_ref, cwt_ref, cb_ref, out_ref):
    out_ref[:, :] = (
        jnp.dot(emb_ref[:, :], cwt_ref[:, :], preferred_element_type=jnp.float32)
        + cb_ref[:, :]
    ).astype(jnp.bfloat16)


def _project_table(emb_table, conv_w, conv_b):
    return pl.pallas_call(
        _proj_body,
        out_shape=jax.ShapeDtypeStruct((VOCAB, NFILT), jnp.bfloat16),
    )(emb_table, conv_w.T, conv_b.reshape(1, NFILT))


_TPS = VOCAB // _NS


def _enc_body(table_hbm, idx_hbm, out_hbm, tab_sh, idx_all, rows_v, out_v,
              sem0, sem1):
    sid = lax.axis_index("s")
    wid = sid * _NC + lax.axis_index("c")
    sems = (sem0, sem1)

    pltpu.sync_copy(table_hbm.at[pl.ds(sid * _TPS, _TPS)],
                    tab_sh.at[pl.ds(sid * _TPS, _TPS)])
    plsc.subcore_barrier()

    pltpu.sync_copy(idx_hbm.at[pl.ds(wid * _PW * SEQ, _PW * SEQ)], idx_all)

    def fire(g, buf):
        pltpu.async_copy(
            tab_sh.at[idx_all.at[pl.ds(g * _CHROWS, _CHROWS)]],
            rows_v.at[buf],
            sems[buf],
        )

    def drain(buf):
        pltpu.make_async_copy(
            tab_sh.at[idx_all.at[pl.ds(0, _CHROWS)]],
            rows_v.at[buf],
            sems[buf],
        ).wait()

    def compute(g, buf):
        def node_body(n, c2):
            acc = jnp.zeros((2 * 16,), jnp.bfloat16)
            for s in range(SEQ):
                acc = jnp.maximum(acc, rows_v[buf, n * SEQ + s, :])
            out_v[g * _CH + n, :] = acc
            return c2

        lax.fori_loop(0, _CH, node_body, 0)

    fire(0, 0)

    def pair_body(k, carry):
        g0 = 2 * k
        fire(g0 + 1, 1)
        drain(0)
        compute(g0, 0)

        @pl.when(k < _NCHUNK // 2 - 1)
        def _():
            fire(g0 + 2, 0)

        drain(1)
        compute(g0 + 1, 1)
        return carry

    lax.fori_loop(0, _NCHUNK // 2, pair_body, 0)
    pltpu.sync_copy(out_v, out_hbm.at[pl.ds(wid * _PW, _PW)])


def _encode_sc(table, idx_flat):
    mesh = plsc.VectorSubcoreMesh(core_axis_name="c", subcore_axis_name="s")
    k = functools.partial(
        pl.kernel,
        mesh=mesh,
        compiler_params=pltpu.CompilerParams(use_tc_tiling_on_sc=False),
        out_type=jax.ShapeDtypeStruct((_NNP, NFILT), jnp.bfloat16),
        scratch_types=[
            pltpu.VMEM_SHARED((VOCAB, NFILT), jnp.bfloat16),
            pltpu.VMEM((_PW * SEQ,), jnp.int32),
            pltpu.VMEM((2, _CHROWS, NFILT), jnp.bfloat16),
            pltpu.VMEM((_PW, NFILT), jnp.bfloat16),
            pltpu.SemaphoreType.DMA,
            pltpu.SemaphoreType.DMA,
        ],
    )(_enc_body)
    return k(table, idx_flat)


_BM = 400
_NBLK = V // _BM


def _adj_body(adj_ref, el_ref, er_ref, vf_ref, w2v0_ref, vfw0_ref, b0_ref,
              w2v1_ref, b2v1_ref, vf1_ref, bvf1_ref, g_ref,
              rw0_ref, rb0_ref, rw1_ref, rb1_ref,
              out_ref, sup_ref, h_ref, s_ref):
    i = pl.program_id(0)

    @pl.when(i == 0)
    def _():
        el = el_ref[:, :].astype(jnp.float32)
        er = er_ref[:, :].astype(jnp.float32)
        xsi = jnp.concatenate([el * er, jnp.abs(el - er)], axis=1)
        sup_s = jnp.dot(xsi, w2v0_ref[:, :], preferred_element_type=jnp.float32)
        sup_v = jnp.dot(vf_ref[:, :], vfw0_ref[:, :], preferred_element_type=jnp.float32)
        sup_ref[:, :] = jnp.concatenate([sup_s, sup_v], axis=1)

    blk = adj_ref[:, :]
    h_ref[pl.ds(i * _BM, _BM), :] = jnp.maximum(
        jnp.dot(blk, sup_ref[:, :], preferred_element_type=jnp.float32)
        + b0_ref[:, :],
        0.0,
    )
    csum = jnp.sum(blk, axis=0, keepdims=True)

    @pl.when(i == 0)
    def _():
        s_ref[:, :] = csum

    @pl.when(i > 0)
    def _():
        s_ref[:, :] = s_ref[:, :] + csum

    @pl.when(i == _NBLK - 1)
    def _():
        z = jnp.dot(s_ref[:, :], h_ref[:, :], preferred_element_type=jnp.float32) / V
        zs = jnp.dot(z[:, :16], w2v1_ref[:, :], preferred_element_type=jnp.float32) + b2v1_ref[:, :]
        zv = jnp.dot(z[:, 16:], vf1_ref[:, :], preferred_element_type=jnp.float32) + bvf1_ref[:, :]
        x = jnp.concatenate([zs, zv, g_ref[:, :]], axis=1)
        hh = jnp.maximum(
            jnp.dot(x, rw0_ref[:, :], preferred_element_type=jnp.float32) + rb0_ref[:, :],
            0.0,
        )
        pre = jnp.dot(hh, rw1_ref[:, :], preferred_element_type=jnp.float32) + rb1_ref[:, :]
        out_ref[:, :] = jax.nn.sigmoid(pre)


def _adj_pass(adj, encl, encr, v_feature, w2v_w0, vf_w0, b0,
              w2v_w1, w2v_b1, vf_w1, vf_b1, g_feature,
              reg_w0, reg_b0, reg_w1, reg_b1):
    full = lambda shape: pl.BlockSpec(shape, lambda i: tuple(0 for _ in shape))
    return pl.pallas_call(
        _adj_body,
        grid=(_NBLK,),
        in_specs=[
            pl.BlockSpec((_BM, V), lambda i: (i, 0)),
            full((V, NFILT)), full((V, NFILT)),
            full((V, 128)),
            full((2 * NFILT, 16)), full((128, 16)), full((1, NFILT)),
            full((16, 16)), full((1, 16)), full((16, 16)), full((1, 16)),
            full((1, NFILT)),
            full((2 * 16 + NFEAT_G, 16)), full((1, 16)), full((16, 1)), full((1, 1)),
        ],
        out_specs=pl.BlockSpec((1, 1), lambda i: (0, 0)),
        out_shape=jax.ShapeDtypeStruct((1, 1), jnp.float32),
        scratch_shapes=[
            pltpu.VMEM((V, NFILT), jnp.float32),
            pltpu.VMEM((V, NFILT), jnp.float32),
            pltpu.VMEM((1, V), jnp.float32),
        ],
    )(adj, encl, encr, v_feature, w2v_w0, vf_w0, b0,
      w2v_w1, w2v_b1.reshape(1, -1), vf_w1, vf_b1.reshape(1, -1),
      g_feature.reshape(1, -1), reg_w0, reg_b0.reshape(1, -1),
      reg_w1, reg_b1.reshape(1, -1))


def kernel(x_l, x_r, v_feature, adj, g_feature, emb_table, conv_w, conv_b,
           w2v_w0, w2v_b0, w2v_w1, w2v_b1, vf_w0, vf_b0, vf_w1, vf_b1,
           reg_w0, reg_b0, reg_w1, reg_b1):
    table = _project_table(emb_table, conv_w, conv_b)

    idx = jnp.concatenate([x_l, x_r], axis=0)
    idx = jnp.pad(idx, ((0, _NNP - _NN), (0, 0)))
    idx_flat = idx.reshape(_NNP * SEQ)

    enc = _encode_sc(table, idx_flat)

    b0 = jnp.concatenate([w2v_b0, vf_b0]).reshape(1, NFILT)
    out = _adj_pass(adj, enc[:V], enc[V:_NN], v_feature, w2v_w0, vf_w0, b0,
                    w2v_w1, w2v_b1, vf_w1, vf_b1, g_feature,
                    reg_w0, reg_b0, reg_w1, reg_b1)
    return out.reshape(1)


# speedup vs baseline: 12.5183x; 1.0148x over previous
import functools

import jax
import jax.numpy as jnp
from jax import lax
from jax.experimental import pallas as pl
from jax.experimental.pallas import tpu as pltpu
from jax.experimental.pallas import tpu_sc as plsc

V = 10000
SEQ = 50
VOCAB = 30000
NFILT = 32
NFEAT_G = 32

_NC = 2
_NS = 16
_NW = _NC * _NS
_NN = 2 * V
_NNP = 20480
_PW = _NNP // _NW
_CH = 32
_NCHUNK = _PW // _CH
_CHROWS = _CH * SEQ


def _proj_body(emb_ref, cwt_ref, cb_ref, out_ref):
    out_ref[:, :] = (
        jnp.dot(emb_ref[:, :], cwt_ref[:, :], preferred_element_type=jnp.float32)
        + cb_ref[:, :]
    ).astype(jnp.bfloat16)


def _project_table(emb_table, conv_w, conv_b):
    return pl.pallas_call(
        _proj_body,
        out_shape=jax.ShapeDtypeStruct((VOCAB, NFILT), jnp.bfloat16),
    )(emb_table, conv_w.T, conv_b.reshape(1, NFILT))


_TPS = VOCAB // _NS


def _enc_body(table_hbm, idx_hbm, out_hbm, tab_sh, idx_all, rows_v, out_v,
              sem0, sem1):
    sid = lax.axis_index("s")
    wid = sid * _NC + lax.axis_index("c")
    sems = (sem0, sem1)

    fill = pltpu.make_async_copy(table_hbm.at[pl.ds(sid * _TPS, _TPS)],
                                 tab_sh.at[pl.ds(sid * _TPS, _TPS)], sem0)
    fill.start()
    pltpu.sync_copy(idx_hbm.at[pl.ds(wid * _PW * SEQ, _PW * SEQ)], idx_all)
    fill.wait()
    plsc.subcore_barrier()

    def fire(g, buf):
        pltpu.async_copy(
            tab_sh.at[idx_all.at[pl.ds(g * _CHROWS, _CHROWS)]],
            rows_v.at[buf],
            sems[buf],
        )

    def drain(buf):
        pltpu.make_async_copy(
            tab_sh.at[idx_all.at[pl.ds(0, _CHROWS)]],
            rows_v.at[buf],
            sems[buf],
        ).wait()

    def compute(g, buf):
        def node_body(n, c2):
            acc = jnp.zeros((2 * 16,), jnp.bfloat16)
            for s in range(SEQ):
                acc = jnp.maximum(acc, rows_v[buf, n * SEQ + s, :])
            out_v[g * _CH + n, :] = acc
            return c2

        lax.fori_loop(0, _CH, node_body, 0)

    fire(0, 0)

    def pair_body(k, carry):
        g0 = 2 * k
        fire(g0 + 1, 1)
        drain(0)
        compute(g0, 0)

        @pl.when(k < _NCHUNK // 2 - 1)
        def _():
            fire(g0 + 2, 0)

        drain(1)
        compute(g0 + 1, 1)
        return carry

    lax.fori_loop(0, _NCHUNK // 2, pair_body, 0)
    pltpu.sync_copy(out_v, out_hbm.at[pl.ds(wid * _PW, _PW)])


def _encode_sc(table, idx_flat):
    mesh = plsc.VectorSubcoreMesh(core_axis_name="c", subcore_axis_name="s")
    k = functools.partial(
        pl.kernel,
        mesh=mesh,
        compiler_params=pltpu.CompilerParams(use_tc_tiling_on_sc=False),
        out_type=jax.ShapeDtypeStruct((_NNP, NFILT), jnp.bfloat16),
        scratch_types=[
            pltpu.VMEM_SHARED((VOCAB, NFILT), jnp.bfloat16),
            pltpu.VMEM((_PW * SEQ,), jnp.int32),
            pltpu.VMEM((2, _CHROWS, NFILT), jnp.bfloat16),
            pltpu.VMEM((_PW, NFILT), jnp.bfloat16),
            pltpu.SemaphoreType.DMA,
            pltpu.SemaphoreType.DMA,
        ],
    )(_enc_body)
    return k(table, idx_flat)


_BM = 400
_NBLK = V // _BM


def _adj_body(adj_ref, el_ref, er_ref, vf_ref, w2v0_ref, vfw0_ref, b0_ref,
              w2v1_ref, b2v1_ref, vf1_ref, bvf1_ref, g_ref,
              rw0_ref, rb0_ref, rw1_ref, rb1_ref,
              out_ref, sup_ref, h_ref, s_ref):
    i = pl.program_id(0)

    @pl.when(i == 0)
    def _():
        el = el_ref[:, :].astype(jnp.float32)
        er = er_ref[:, :].astype(jnp.float32)
        xsi = jnp.concatenate([el * er, jnp.abs(el - er)], axis=1)
        sup_s = jnp.dot(xsi, w2v0_ref[:, :], preferred_element_type=jnp.float32)
        sup_v = jnp.dot(vf_ref[:, :], vfw0_ref[:, :], preferred_element_type=jnp.float32)
        sup_ref[:, :] = jnp.concatenate([sup_s, sup_v], axis=1)

    blk = adj_ref[:, :]
    h_ref[pl.ds(i * _BM, _BM), :] = jnp.maximum(
        jnp.dot(blk, sup_ref[:, :], preferred_element_type=jnp.float32)
        + b0_ref[:, :],
        0.0,
    )
    csum = jnp.sum(blk, axis=0, keepdims=True)

    @pl.when(i == 0)
    def _():
        s_ref[:, :] = csum

    @pl.when(i > 0)
    def _():
        s_ref[:, :] = s_ref[:, :] + csum

    @pl.when(i == _NBLK - 1)
    def _():
        z = jnp.dot(s_ref[:, :], h_ref[:, :], preferred_element_type=jnp.float32) / V
        zs = jnp.dot(z[:, :16], w2v1_ref[:, :], preferred_element_type=jnp.float32) + b2v1_ref[:, :]
        zv = jnp.dot(z[:, 16:], vf1_ref[:, :], preferred_element_type=jnp.float32) + bvf1_ref[:, :]
        x = jnp.concatenate([zs, zv, g_ref[:, :]], axis=1)
        hh = jnp.maximum(
            jnp.dot(x, rw0_ref[:, :], preferred_element_type=jnp.float32) + rb0_ref[:, :],
            0.0,
        )
        pre = jnp.dot(hh, rw1_ref[:, :], preferred_element_type=jnp.float32) + rb1_ref[:, :]
        out_ref[:, :] = jax.nn.sigmoid(pre)


def _adj_pass(adj, encl, encr, v_feature, w2v_w0, vf_w0, b0,
              w2v_w1, w2v_b1, vf_w1, vf_b1, g_feature,
              reg_w0, reg_b0, reg_w1, reg_b1):
    full = lambda shape: pl.BlockSpec(shape, lambda i: tuple(0 for _ in shape))
    return pl.pallas_call(
        _adj_body,
        grid=(_NBLK,),
        in_specs=[
            pl.BlockSpec((_BM, V), lambda i: (i, 0)),
            full((V, NFILT)), full((V, NFILT)),
            full((V, 128)),
            full((2 * NFILT, 16)), full((128, 16)), full((1, NFILT)),
            full((16, 16)), full((1, 16)), full((16, 16)), full((1, 16)),
            full((1, NFILT)),
            full((2 * 16 + NFEAT_G, 16)), full((1, 16)), full((16, 1)), full((1, 1)),
        ],
        out_specs=pl.BlockSpec((1, 1), lambda i: (0, 0)),
        out_shape=jax.ShapeDtypeStruct((1, 1), jnp.float32),
        scratch_shapes=[
            pltpu.VMEM((V, NFILT), jnp.float32),
            pltpu.VMEM((V, NFILT), jnp.float32),
            pltpu.VMEM((1, V), jnp.float32),
        ],
    )(adj, encl, encr, v_feature, w2v_w0, vf_w0, b0,
      w2v_w1, w2v_b1.reshape(1, -1), vf_w1, vf_b1.reshape(1, -1),
      g_feature.reshape(1, -1), reg_w0, reg_b0.reshape(1, -1),
      reg_w1, reg_b1.reshape(1, -1))


def kernel(x_l, x_r, v_feature, adj, g_feature, emb_table, conv_w, conv_b,
           w2v_w0, w2v_b0, w2v_w1, w2v_b1, vf_w0, vf_b0, vf_w1, vf_b1,
           reg_w0, reg_b0, reg_w1, reg_b1):
    table = _project_table(emb_table, conv_w, conv_b)

    idx = jnp.concatenate([x_l, x_r], axis=0)
    idx = jnp.pad(idx, ((0, _NNP - _NN), (0, 0)))
    idx_flat = idx.reshape(_NNP * SEQ)

    enc = _encode_sc(table, idx_flat)

    b0 = jnp.concatenate([w2v_b0, vf_b0]).reshape(1, NFILT)
    out = _adj_pass(adj, enc[:V], enc[V:_NN], v_feature, w2v_w0, vf_w0, b0,
                    w2v_w1, w2v_b1, vf_w1, vf_b1, g_feature,
                    reg_w0, reg_b0, reg_w1, reg_b1)
    return out.reshape(1)


# speedup vs baseline: 12.6681x; 1.0120x over previous
import functools

import jax
import jax.numpy as jnp
from jax import lax
from jax.experimental import pallas as pl
from jax.experimental.pallas import tpu as pltpu
from jax.experimental.pallas import tpu_sc as plsc

V = 10000
SEQ = 50
VOCAB = 30000
NFILT = 32
NFEAT_G = 32

_NC = 2
_NS = 16
_NW = _NC * _NS
_NN = 2 * V
_NNP = 20480
_PW = _NNP // _NW
_CH = 32
_NCHUNK = _PW // _CH
_CHROWS = _CH * SEQ


def _proj_body(emb_ref, cwt_ref, cb_ref, out_ref):
    out_ref[:, :] = (
        jnp.dot(emb_ref[:, :], cwt_ref[:, :], preferred_element_type=jnp.float32)
        + cb_ref[:, :]
    ).astype(jnp.bfloat16)


def _project_table(emb_table, conv_w, conv_b):
    return pl.pallas_call(
        _proj_body,
        out_shape=jax.ShapeDtypeStruct((VOCAB, NFILT), jnp.bfloat16),
    )(emb_table, conv_w.T, conv_b.reshape(1, NFILT))


_TPS = VOCAB // _NS


def _enc_body(table_hbm, idx_hbm, out_hbm, tab_sh, idx_all, rows_v, out_v,
              sem0, sem1):
    sid = lax.axis_index("s")
    wid = sid * _NC + lax.axis_index("c")
    sems = (sem0, sem1)

    fill = pltpu.make_async_copy(table_hbm.at[pl.ds(sid * _TPS, _TPS)],
                                 tab_sh.at[pl.ds(sid * _TPS, _TPS)], sem0)
    fill.start()
    pltpu.sync_copy(idx_hbm.at[pl.ds(wid * _PW * SEQ, _PW * SEQ)], idx_all)
    fill.wait()
    plsc.subcore_barrier()

    def fire(g, buf):
        pltpu.async_copy(
            tab_sh.at[idx_all.at[pl.ds(g * _CHROWS, _CHROWS)]],
            rows_v.at[buf],
            sems[buf],
        )

    def drain(buf):
        pltpu.make_async_copy(
            tab_sh.at[idx_all.at[pl.ds(0, _CHROWS)]],
            rows_v.at[buf],
            sems[buf],
        ).wait()

    def compute(g, buf):
        def node_body(n, c2):
            acc = jnp.zeros((2 * 16,), jnp.bfloat16)
            for s in range(SEQ):
                acc = jnp.maximum(acc, rows_v[buf, n * SEQ + s, :])
            out_v[g * _CH + n, :] = acc
            return c2

        lax.fori_loop(0, _CH, node_body, 0)

    fire(0, 0)

    def pair_body(k, carry):
        g0 = 2 * k
        fire(g0 + 1, 1)
        drain(0)
        compute(g0, 0)

        @pl.when(k < _NCHUNK // 2 - 1)
        def _():
            fire(g0 + 2, 0)

        drain(1)
        compute(g0 + 1, 1)
        return carry

    lax.fori_loop(0, _NCHUNK // 2, pair_body, 0)
    pltpu.sync_copy(out_v, out_hbm.at[pl.ds(wid * _PW, _PW)])


def _encode_sc(table, idx_flat):
    mesh = plsc.VectorSubcoreMesh(core_axis_name="c", subcore_axis_name="s")
    k = functools.partial(
        pl.kernel,
        mesh=mesh,
        compiler_params=pltpu.CompilerParams(use_tc_tiling_on_sc=False),
        out_type=jax.ShapeDtypeStruct((_NNP, NFILT), jnp.bfloat16),
        scratch_types=[
            pltpu.VMEM_SHARED((VOCAB, NFILT), jnp.bfloat16),
            pltpu.VMEM((_PW * SEQ,), jnp.int32),
            pltpu.VMEM((2, _CHROWS, NFILT), jnp.bfloat16),
            pltpu.VMEM((_PW, NFILT), jnp.bfloat16),
            pltpu.SemaphoreType.DMA,
            pltpu.SemaphoreType.DMA,
        ],
    )(_enc_body)
    return k(table, idx_flat)


_BM = 400
_NBLK = V // _BM


def _adj_body(adj_ref, el_ref, er_ref, vf_ref, w2v0_ref, vfw0_ref, b0_ref,
              w2v1_ref, b2v1_ref, vf1_ref, bvf1_ref, g_ref,
              rw0_ref, rb0_ref, rw1_ref, rb1_ref,
              out_ref, sup_ref, h_ref, s_ref):
    i = pl.program_id(0)

    @pl.when(i == 0)
    def _():
        el = el_ref[:, :].astype(jnp.float32)
        er = er_ref[:, :].astype(jnp.float32)
        xsi = jnp.concatenate([el * er, jnp.abs(el - er)], axis=1)
        sup_s = jnp.dot(xsi, w2v0_ref[:, :], preferred_element_type=jnp.float32)
        sup_v = jnp.dot(vf_ref[:, :], vfw0_ref[:, :], preferred_element_type=jnp.float32)
        sup_ref[:, :] = jnp.concatenate([sup_s, sup_v], axis=1)

    blk = adj_ref[:, :]
    h_ref[pl.ds(i * _BM, _BM), :] = jnp.maximum(
        jnp.dot(blk, sup_ref[:, :], preferred_element_type=jnp.float32)
        + b0_ref[:, :],
        0.0,
    )
    csum = jnp.sum(blk, axis=0, keepdims=True)

    @pl.when(i == 0)
    def _():
        s_ref[:, :] = csum

    @pl.when(i > 0)
    def _():
        s_ref[:, :] = s_ref[:, :] + csum

    @pl.when(i == _NBLK - 1)
    def _():
        z = jnp.dot(s_ref[:, :], h_ref[:, :], preferred_element_type=jnp.float32) / V
        zs = jnp.dot(z[:, :16], w2v1_ref[:, :], preferred_element_type=jnp.float32) + b2v1_ref[:, :]
        zv = jnp.dot(z[:, 16:], vf1_ref[:, :], preferred_element_type=jnp.float32) + bvf1_ref[:, :]
        x = jnp.concatenate([zs, zv, g_ref[:, :]], axis=1)
        hh = jnp.maximum(
            jnp.dot(x, rw0_ref[:, :], preferred_element_type=jnp.float32) + rb0_ref[:, :],
            0.0,
        )
        pre = jnp.dot(hh, rw1_ref[:, :], preferred_element_type=jnp.float32) + rb1_ref[:, :]
        out_ref[:, :] = jax.nn.sigmoid(pre)


def _adj_pass(adj, encl, encr, v_feature, w2v_w0, vf_w0, b0,
              w2v_w1, w2v_b1, vf_w1, vf_b1, g_feature,
              reg_w0, reg_b0, reg_w1, reg_b1):
    full = lambda shape: pl.BlockSpec(shape, lambda i: tuple(0 for _ in shape))
    return pl.pallas_call(
        _adj_body,
        grid=(_NBLK,),
        in_specs=[
            pl.BlockSpec((_BM, V), lambda i: (i, 0)),
            pl.BlockSpec((V, NFILT), lambda i: (0, 0)),
            pl.BlockSpec((V, NFILT), lambda i: (1, 0)),
            full((V, 128)),
            full((2 * NFILT, 16)), full((128, 16)), full((1, NFILT)),
            full((16, 16)), full((1, 16)), full((16, 16)), full((1, 16)),
            full((1, NFILT)),
            full((2 * 16 + NFEAT_G, 16)), full((1, 16)), full((16, 1)), full((1, 1)),
        ],
        out_specs=pl.BlockSpec((1, 1), lambda i: (0, 0)),
        out_shape=jax.ShapeDtypeStruct((1, 1), jnp.float32),
        scratch_shapes=[
            pltpu.VMEM((V, NFILT), jnp.float32),
            pltpu.VMEM((V, NFILT), jnp.float32),
            pltpu.VMEM((1, V), jnp.float32),
        ],
    )(adj, encl, encr, v_feature, w2v_w0, vf_w0, b0,
      w2v_w1, w2v_b1.reshape(1, -1), vf_w1, vf_b1.reshape(1, -1),
      g_feature.reshape(1, -1), reg_w0, reg_b0.reshape(1, -1),
      reg_w1, reg_b1.reshape(1, -1))


def kernel(x_l, x_r, v_feature, adj, g_feature, emb_table, conv_w, conv_b,
           w2v_w0, w2v_b0, w2v_w1, w2v_b1, vf_w0, vf_b0, vf_w1, vf_b1,
           reg_w0, reg_b0, reg_w1, reg_b1):
    table = _project_table(emb_table, conv_w, conv_b)

    idx = jnp.concatenate([x_l, x_r], axis=0)
    idx = jnp.pad(idx, ((0, _NNP - _NN), (0, 0)))
    idx_flat = idx.reshape(_NNP * SEQ)

    enc = _encode_sc(table, idx_flat)

    b0 = jnp.concatenate([w2v_b0, vf_b0]).reshape(1, NFILT)
    out = _adj_pass(adj, enc, enc, v_feature, w2v_w0, vf_w0, b0,
                    w2v_w1, w2v_b1, vf_w1, vf_b1, g_feature,
                    reg_w0, reg_b0, reg_w1, reg_b1)
    return out.reshape(1)


# speedup vs baseline: 12.7739x; 1.0084x over previous
import functools

import jax
import jax.numpy as jnp
from jax import lax
from jax.experimental import pallas as pl
from jax.experimental.pallas import tpu as pltpu
from jax.experimental.pallas import tpu_sc as plsc

V = 10000
SEQ = 50
VOCAB = 30000
NFILT = 32
NFEAT_G = 32

_NC = 2
_NS = 16
_NW = _NC * _NS
_NN = 2 * V
_NNP = 20480
_PW = _NNP // _NW
_CH = 32
_NCHUNK = _PW // _CH
_CHROWS = _CH * SEQ


def _proj_body(emb_ref, cwt_ref, cb_ref, out_ref):
    out_ref[:, :] = (
        jnp.dot(emb_ref[:, :], cwt_ref[:, :], preferred_element_type=jnp.float32)
        + cb_ref[:, :]
    ).astype(jnp.bfloat16)


def _project_table(emb_table, conv_w, conv_b):
    return pl.pallas_call(
        _proj_body,
        out_shape=jax.ShapeDtypeStruct((VOCAB, NFILT), jnp.bfloat16),
    )(emb_table, conv_w.T, conv_b.reshape(1, NFILT))


_TPS = VOCAB // _NS


def _enc_body(table_hbm, idx_hbm, out_hbm, tab_sh, idx_all, rows_v, out_v,
              sem0, sem1):
    sid = lax.axis_index("s")
    wid = sid * _NC + lax.axis_index("c")
    sems = (sem0, sem1)

    fill = pltpu.make_async_copy(table_hbm.at[pl.ds(sid * _TPS, _TPS)],
                                 tab_sh.at[pl.ds(sid * _TPS, _TPS)], sem0)
    fill.start()
    pltpu.sync_copy(idx_hbm.at[pl.ds(wid * _PW * SEQ, _PW * SEQ)], idx_all)
    fill.wait()
    plsc.subcore_barrier()

    def fire(g, buf):
        pltpu.async_copy(
            tab_sh.at[idx_all.at[pl.ds(g * _CHROWS, _CHROWS)]],
            rows_v.at[buf],
            sems[buf],
        )

    def drain(buf):
        pltpu.make_async_copy(
            tab_sh.at[idx_all.at[pl.ds(0, _CHROWS)]],
            rows_v.at[buf],
            sems[buf],
        ).wait()

    def compute(g, buf):
        def node_body(n, c2):
            acc = jnp.zeros((2 * 16,), jnp.bfloat16)
            for s in range(SEQ):
                acc = jnp.maximum(acc, rows_v[buf, n * SEQ + s, :])
            out_v[g * _CH + n, :] = acc
            return c2

        lax.fori_loop(0, _CH, node_body, 0)

    fire(0, 0)

    def pair_body(k, carry):
        g0 = 2 * k
        fire(g0 + 1, 1)
        drain(0)
        compute(g0, 0)

        @pl.when(k < _NCHUNK // 2 - 1)
        def _():
            fire(g0 + 2, 0)

        drain(1)
        compute(g0 + 1, 1)
        return carry

    lax.fori_loop(0, _NCHUNK // 2, pair_body, 0)
    pltpu.sync_copy(out_v, out_hbm.at[pl.ds(wid * _PW, _PW)])


def _encode_sc(table, idx_flat):
    mesh = plsc.VectorSubcoreMesh(core_axis_name="c", subcore_axis_name="s")
    k = functools.partial(
        pl.kernel,
        mesh=mesh,
        compiler_params=pltpu.CompilerParams(use_tc_tiling_on_sc=False),
        out_type=jax.ShapeDtypeStruct((_NNP, NFILT), jnp.bfloat16),
        scratch_types=[
            pltpu.VMEM_SHARED((VOCAB, NFILT), jnp.bfloat16),
            pltpu.VMEM((_PW * SEQ,), jnp.int32),
            pltpu.VMEM((2, _CHROWS, NFILT), jnp.bfloat16),
            pltpu.VMEM((_PW, NFILT), jnp.bfloat16),
            pltpu.SemaphoreType.DMA,
            pltpu.SemaphoreType.DMA,
        ],
    )(_enc_body)
    return k(table, idx_flat)


_BM = 400
_NBLK = V // _BM


_NSTR = 2
_BSTR = _BM // _NSTR


def _adj_body(adj_hbm, el_ref, er_ref, vf_ref, w2v0_ref, vfw0_ref, b0_ref,
              w2v1_ref, b2v1_ref, vf1_ref, bvf1_ref, g_ref,
              rw0_ref, rb0_ref, rw1_ref, rb1_ref,
              out_ref, sup_ref, h_ref, s_ref, buf, sem0, sem1):
    i = pl.program_id(0)
    sems = (sem0, sem1)

    def stream(blk_i, slot, q):
        return pltpu.make_async_copy(
            adj_hbm.at[pl.ds(blk_i * _BM + q * _BSTR, _BSTR), :],
            buf.at[slot, pl.ds(q * _BSTR, _BSTR), :],
            sems[q],
        )

    def start_all(blk_i, slot):
        for q in range(_NSTR):
            stream(blk_i, slot, q).start()

    def wait_all(slot):
        for q in range(_NSTR):
            stream(0, slot, q).wait()

    @pl.when(i == 0)
    def _():
        el = el_ref[:, :].astype(jnp.float32)
        er = er_ref[:, :].astype(jnp.float32)
        xsi = jnp.concatenate([el * er, jnp.abs(el - er)], axis=1)
        sup_s = jnp.dot(xsi, w2v0_ref[:, :], preferred_element_type=jnp.float32)
        sup_v = jnp.dot(vf_ref[:, :], vfw0_ref[:, :], preferred_element_type=jnp.float32)
        sup_ref[:, :] = jnp.concatenate([sup_s, sup_v], axis=1)

    @pl.when(i == 0)
    def _():
        start_all(0, 0)

    @pl.when(i + 1 < _NBLK)
    def _():
        start_all(i + 1, (i + 1) % 2)

    def consume(slot):
        wait_all(slot)
        blk = buf[slot]
        h_ref[pl.ds(i * _BM, _BM), :] = jnp.maximum(
            jnp.dot(blk, sup_ref[:, :], preferred_element_type=jnp.float32)
            + b0_ref[:, :],
            0.0,
        )
        csum = jnp.sum(blk, axis=0, keepdims=True)

        @pl.when(i == 0)
        def _():
            s_ref[:, :] = csum

        @pl.when(i > 0)
        def _():
            s_ref[:, :] = s_ref[:, :] + csum

    @pl.when(i % 2 == 0)
    def _():
        consume(0)

    @pl.when(i % 2 == 1)
    def _():
        consume(1)

    @pl.when(i == _NBLK - 1)
    def _():
        z = jnp.dot(s_ref[:, :], h_ref[:, :], preferred_element_type=jnp.float32) / V
        zs = jnp.dot(z[:, :16], w2v1_ref[:, :], preferred_element_type=jnp.float32) + b2v1_ref[:, :]
        zv = jnp.dot(z[:, 16:], vf1_ref[:, :], preferred_element_type=jnp.float32) + bvf1_ref[:, :]
        x = jnp.concatenate([zs, zv, g_ref[:, :]], axis=1)
        hh = jnp.maximum(
            jnp.dot(x, rw0_ref[:, :], preferred_element_type=jnp.float32) + rb0_ref[:, :],
            0.0,
        )
        pre = jnp.dot(hh, rw1_ref[:, :], preferred_element_type=jnp.float32) + rb1_ref[:, :]
        out_ref[:, :] = jax.nn.sigmoid(pre)


def _adj_pass(adj, encl, encr, v_feature, w2v_w0, vf_w0, b0,
              w2v_w1, w2v_b1, vf_w1, vf_b1, g_feature,
              reg_w0, reg_b0, reg_w1, reg_b1):
    full = lambda shape: pl.BlockSpec(shape, lambda i: tuple(0 for _ in shape))
    return pl.pallas_call(
        _adj_body,
        grid=(_NBLK,),
        in_specs=[
            pl.BlockSpec(memory_space=pl.ANY),
            pl.BlockSpec((V, NFILT), lambda i: (0, 0)),
            pl.BlockSpec((V, NFILT), lambda i: (1, 0)),
            full((V, 128)),
            full((2 * NFILT, 16)), full((128, 16)), full((1, NFILT)),
            full((16, 16)), full((1, 16)), full((16, 16)), full((1, 16)),
            full((1, NFILT)),
            full((2 * 16 + NFEAT_G, 16)), full((1, 16)), full((16, 1)), full((1, 1)),
        ],
        out_specs=pl.BlockSpec((1, 1), lambda i: (0, 0)),
        out_shape=jax.ShapeDtypeStruct((1, 1), jnp.float32),
        scratch_shapes=[
            pltpu.VMEM((V, NFILT), jnp.float32),
            pltpu.VMEM((V, NFILT), jnp.float32),
            pltpu.VMEM((1, V), jnp.float32),
            pltpu.VMEM((2, _BM, V), jnp.float32),
            pltpu.SemaphoreType.DMA,
            pltpu.SemaphoreType.DMA,
        ],
    )(adj, encl, encr, v_feature, w2v_w0, vf_w0, b0,
      w2v_w1, w2v_b1.reshape(1, -1), vf_w1, vf_b1.reshape(1, -1),
      g_feature.reshape(1, -1), reg_w0, reg_b0.reshape(1, -1),
      reg_w1, reg_b1.reshape(1, -1))


def kernel(x_l, x_r, v_feature, adj, g_feature, emb_table, conv_w, conv_b,
           w2v_w0, w2v_b0, w2v_w1, w2v_b1, vf_w0, vf_b0, vf_w1, vf_b1,
           reg_w0, reg_b0, reg_w1, reg_b1):
    table = _project_table(emb_table, conv_w, conv_b)

    idx = jnp.concatenate([x_l, x_r], axis=0)
    idx = jnp.pad(idx, ((0, _NNP - _NN), (0, 0)))
    idx_flat = idx.reshape(_NNP * SEQ)

    enc = _encode_sc(table, idx_flat)

    b0 = jnp.concatenate([w2v_b0, vf_b0]).reshape(1, NFILT)
    out = _adj_pass(adj, enc, enc, v_feature, w2v_w0, vf_w0, b0,
                    w2v_w1, w2v_b1, vf_w1, vf_b1, g_feature,
                    reg_w0, reg_b0, reg_w1, reg_b1)
    return out.reshape(1)
